# baseline (speedup 1.0000x reference)
import jax
import jax.numpy as jnp
from jax.experimental import pallas as pl

_B = 16384
_Q = 512
_E = 64
_BLK = 1024


def _fused_kernel(x_ref, kdw_ref, cbw_ref, out_ref):
    x = x_ref[...]

    kd = jax.lax.dot_general(
        x, kdw_ref[...], (((1,), (1,)), ((), ())),
        preferred_element_type=jnp.float32,
    )
    kd_max = jnp.max(kd, axis=-1, keepdims=True)
    col = jax.lax.broadcasted_iota(jnp.int32, kd.shape, 1)
    k_idx = jnp.min(jnp.where(kd == kd_max, col, _E), axis=-1, keepdims=True)
    k = k_idx + 1

    ub = jax.lax.bitcast_convert_type(x, jnp.uint32)
    topbit = jnp.uint32(0x80000000)
    ub = jnp.where(ub == topbit, jnp.uint32(0), ub)
    u = jnp.where(ub >= topbit, ~ub, ub | topbit)

    zero_col = jnp.zeros_like(u[:, :1])
    ones_q = jnp.ones((_Q, 1), jnp.float32)
    kf = k.astype(jnp.float32)

    def cond_fn(state):
        bit, _, _, ndone = state
        return (bit >= 0) & (ndone > 0)

    def body_fn(state):
        bit, p, hitcand, _ = state
        cand = p | (jnp.uint32(1) << jnp.uint32(bit))
        mf = jnp.where(u >= cand, 1.0, 0.0)
        c = jax.lax.dot_general(
            mf, ones_q, (((1,), (0,)), ((), ())),
            preferred_element_type=jnp.float32)
        p = jnp.where(c >= kf, cand, p)
        hit = (c == kf) & (hitcand == 0)
        hitcand = jnp.where(hit, cand, hitcand)
        ndone = jnp.sum(jnp.where(hitcand == 0, 1.0, 0.0))
        return bit - 1, p, hitcand, ndone

    _, p_final, hitcand, ndone = jax.lax.while_loop(
        cond_fn, body_fn, (31, zero_col, zero_col, jnp.float32(1.0)))

    def no_ties(_):
        return (u >= hitcand).astype(jnp.float32)

    def with_ties(_):
        thr = jnp.where(hitcand == 0, p_final, hitcand)
        gt = (u > thr).astype(jnp.int32)
        need = k - jnp.sum(gt, axis=-1, keepdims=True)
        idx = jax.lax.broadcasted_iota(jnp.int32, u.shape, 1)
        eq = (u == thr).astype(jnp.int32)

        def idx_step(i, p):
            cand = p + (1 << (9 - i))
            c = jnp.sum(eq * (idx < cand).astype(jnp.int32),
                        axis=-1, keepdims=True)
            return jnp.where(c <= need, cand, p)

        pidx = jax.lax.fori_loop(0, 10, idx_step, jnp.zeros_like(k))
        tie_mask = (gt + eq * (idx < pidx).astype(jnp.int32))
        exact = (u >= hitcand).astype(jnp.int32)
        return jnp.where(hitcand == 0, tie_mask, exact).astype(jnp.float32)

    k_hot = jax.lax.cond(ndone == 0, no_ties, with_ties, operand=None)

    out_ref[...] = jax.lax.dot_general(
        k_hot, cbw_ref[...], (((1,), (1,)), ((), ())),
        preferred_element_type=jnp.float32,
    )


@jax.jit
def kernel(x, codebook_weight, k_decider_weight):
    grid = (_B // _BLK,)
    return pl.pallas_call(
        _fused_kernel,
        grid=grid,
        in_specs=[
            pl.BlockSpec((_BLK, _Q), lambda i: (i, 0)),
            pl.BlockSpec((_E, _Q), lambda i: (0, 0)),
            pl.BlockSpec((_E, _Q), lambda i: (0, 0)),
        ],
        out_specs=pl.BlockSpec((_BLK, _E), lambda i: (i, 0)),
        out_shape=jax.ShapeDtypeStruct((_B, _E), jnp.float32),
    )(x, k_decider_weight, codebook_weight)


# speedup vs baseline: 1.6414x; 1.6414x over previous
import jax
import jax.numpy as jnp
from jax.experimental import pallas as pl

_B = 16384
_Q = 512
_E = 64
_BLK = 1024


def _fused_kernel(x_ref, kdw_ref, cbw_ref, out_ref):
    xt = x_ref[...].T

    kdt = jax.lax.dot_general(
        kdw_ref[...], xt, (((1,), (0,)), ((), ())),
        preferred_element_type=jnp.float32,
    )
    kd_max = jnp.max(kdt, axis=0, keepdims=True)
    col = jax.lax.broadcasted_iota(jnp.int32, kdt.shape, 0)
    k_idx = jnp.min(jnp.where(kdt == kd_max, col, _E), axis=0, keepdims=True)
    k = k_idx + 1
    kf = k.astype(jnp.float32)

    ub = jax.lax.bitcast_convert_type(xt, jnp.uint32)
    topbit = jnp.uint32(0x80000000)
    ub = jnp.where(ub == topbit, jnp.uint32(0), ub)
    u = jnp.where(ub >= topbit, ~ub, ub | topbit)

    zero_row = jnp.zeros_like(u[:1, :])
    ones_q = jnp.ones((1, _Q), jnp.float32)

    def cond_fn(state):
        bit, _, _, ndone = state
        return (bit >= 0) & (ndone > 0)

    def body_fn(state):
        bit, p, hitcand, _ = state
        cand = p | (jnp.uint32(1) << jnp.uint32(bit))
        mf = jnp.where(u >= cand, 1.0, 0.0)
        c = jax.lax.dot_general(
            ones_q, mf, (((1,), (0,)), ((), ())),
            preferred_element_type=jnp.float32)
        p = jnp.where(c >= kf, cand, p)
        hit = (c == kf) & (hitcand == 0)
        hitcand = jnp.where(hit, cand, hitcand)
        ndone = jnp.sum(jnp.where(hitcand == 0, 1.0, 0.0))
        return bit - 1, p, hitcand, ndone

    _, p_final, hitcand, ndone = jax.lax.while_loop(
        cond_fn, body_fn, (31, zero_row, zero_row, jnp.float32(1.0)))

    def no_ties(_):
        return jnp.where(u >= hitcand, 1.0, 0.0)

    def with_ties(_):
        thr = jnp.where(hitcand == 0, p_final, hitcand)
        gt = jnp.where(u > thr, 1.0, 0.0)
        need = k - jnp.sum(gt, axis=0, keepdims=True).astype(jnp.int32)
        idx = jax.lax.broadcasted_iota(jnp.int32, u.shape, 0)
        eq = (u == thr)

        def idx_step(i, p):
            cand = p + (1 << (9 - i))
            sel = jnp.where(eq & (idx < cand), 1.0, 0.0)
            c = jnp.sum(sel, axis=0, keepdims=True).astype(jnp.int32)
            return jnp.where(c <= need, cand, p)

        pidx = jax.lax.fori_loop(0, 10, idx_step, jnp.zeros_like(k))
        tie_mask = gt + jnp.where(eq & (idx < pidx), 1.0, 0.0)
        exact = jnp.where(u >= hitcand, 1.0, 0.0)
        return jnp.where(hitcand == 0, tie_mask, exact)

    k_hot = jax.lax.cond(ndone == 0, no_ties, with_ties, operand=None)

    out_t = jax.lax.dot_general(
        cbw_ref[...], k_hot, (((1,), (0,)), ((), ())),
        preferred_element_type=jnp.float32,
    )
    out_ref[...] = out_t.T


@jax.jit
def kernel(x, codebook_weight, k_decider_weight):
    grid = (_B // _BLK,)
    return pl.pallas_call(
        _fused_kernel,
        grid=grid,
        in_specs=[
            pl.BlockSpec((_BLK, _Q), lambda i: (i, 0)),
            pl.BlockSpec((_E, _Q), lambda i: (0, 0)),
            pl.BlockSpec((_E, _Q), lambda i: (0, 0)),
        ],
        out_specs=pl.BlockSpec((_BLK, _E), lambda i: (i, 0)),
        out_shape=jax.ShapeDtypeStruct((_B, _E), jnp.float32),
    )(x, k_decider_weight, codebook_weight)


# speedup vs baseline: 2.1119x; 1.2866x over previous
import jax
import jax.numpy as jnp
from jax.experimental import pallas as pl

_B = 16384
_Q = 512
_E = 64
_BLK = 2048


def _fused_kernel(x_ref, kdw_ref, cbw_ref, out_ref):
    xt = x_ref[...].T

    kdt = jax.lax.dot_general(
        kdw_ref[...], xt, (((1,), (0,)), ((), ())),
        preferred_element_type=jnp.float32,
    )
    kd_max = jnp.max(kdt, axis=0, keepdims=True)
    col = jax.lax.broadcasted_iota(jnp.int32, kdt.shape, 0)
    k_idx = jnp.min(jnp.where(kdt == kd_max, col, _E), axis=0, keepdims=True)
    k = k_idx + 1
    kf = k.astype(jnp.float32)

    ub = jax.lax.bitcast_convert_type(xt, jnp.uint32)
    topbit = jnp.uint32(0x80000000)
    ub = jnp.where(ub == topbit, jnp.uint32(0), ub)
    u = jnp.where(ub >= topbit, ~ub, ub | topbit)

    zero_row = jnp.zeros_like(u[:1, :])
    ones_q = jnp.ones((1, _Q), jnp.float32)

    def cond_fn(state):
        bit, _, _, ndone = state
        return (bit >= 0) & (ndone > 0)

    def body_fn(state):
        bit, p, hitcand, _ = state
        cand = p | (jnp.uint32(1) << jnp.uint32(bit))
        mf = jnp.where(u >= cand, 1.0, 0.0)
        c = jax.lax.dot_general(
            ones_q, mf, (((1,), (0,)), ((), ())),
            preferred_element_type=jnp.float32)
        p = jnp.where(c >= kf, cand, p)
        hit = (c == kf) & (hitcand == 0)
        hitcand = jnp.where(hit, cand, hitcand)
        ndone = jnp.sum(jnp.where(hitcand == 0, 1.0, 0.0))
        return bit - 1, p, hitcand, ndone

    _, p_final, hitcand, ndone = jax.lax.while_loop(
        cond_fn, body_fn, (31, zero_row, zero_row, jnp.float32(1.0)))

    def no_ties(_):
        return jnp.where(u >= hitcand, 1.0, 0.0)

    def with_ties(_):
        thr = jnp.where(hitcand == 0, p_final, hitcand)
        gt = jnp.where(u > thr, 1.0, 0.0)
        need = k - jnp.sum(gt, axis=0, keepdims=True).astype(jnp.int32)
        idx = jax.lax.broadcasted_iota(jnp.int32, u.shape, 0)
        eq = (u == thr)

        def idx_step(i, p):
            cand = p + (1 << (9 - i))
            sel = jnp.where(eq & (idx < cand), 1.0, 0.0)
            c = jnp.sum(sel, axis=0, keepdims=True).astype(jnp.int32)
            return jnp.where(c <= need, cand, p)

        pidx = jax.lax.fori_loop(0, 10, idx_step, jnp.zeros_like(k))
        tie_mask = gt + jnp.where(eq & (idx < pidx), 1.0, 0.0)
        exact = jnp.where(u >= hitcand, 1.0, 0.0)
        return jnp.where(hitcand == 0, tie_mask, exact)

    k_hot = jax.lax.cond(ndone == 0, no_ties, with_ties, operand=None)

    out_t = jax.lax.dot_general(
        cbw_ref[...], k_hot, (((1,), (0,)), ((), ())),
        preferred_element_type=jnp.float32,
    )
    out_ref[...] = out_t.T


@jax.jit
def kernel(x, codebook_weight, k_decider_weight):
    grid = (_B // _BLK,)
    return pl.pallas_call(
        _fused_kernel,
        grid=grid,
        in_specs=[
            pl.BlockSpec((_BLK, _Q), lambda i: (i, 0)),
            pl.BlockSpec((_E, _Q), lambda i: (0, 0)),
            pl.BlockSpec((_E, _Q), lambda i: (0, 0)),
        ],
        out_specs=pl.BlockSpec((_BLK, _E), lambda i: (i, 0)),
        out_shape=jax.ShapeDtypeStruct((_B, _E), jnp.float32),
    )(x, k_decider_weight, codebook_weight)


# speedup vs baseline: 2.4765x; 1.1726x over previous
import jax
import jax.numpy as jnp
from jax.experimental import pallas as pl

_B = 16384
_Q = 512
_E = 64
_BLK = 4096


def _fused_kernel(x_ref, kdw_ref, cbw_ref, out_ref):
    xt = x_ref[...].T

    kdt = jax.lax.dot_general(
        kdw_ref[...], xt, (((1,), (0,)), ((), ())),
        preferred_element_type=jnp.float32,
    )
    kd_max = jnp.max(kdt, axis=0, keepdims=True)
    col = jax.lax.broadcasted_iota(jnp.int32, kdt.shape, 0)
    k_idx = jnp.min(jnp.where(kdt == kd_max, col, _E), axis=0, keepdims=True)
    k = k_idx + 1
    kf = k.astype(jnp.float32)

    ub = jax.lax.bitcast_convert_type(xt, jnp.uint32)
    topbit = jnp.uint32(0x80000000)
    ub = jnp.where(ub == topbit, jnp.uint32(0), ub)
    u = jnp.where(ub >= topbit, ~ub, ub | topbit)

    zero_row = jnp.zeros_like(u[:1, :])
    ones_q = jnp.ones((1, _Q), jnp.float32)

    def cond_fn(state):
        bit, _, _, ndone = state
        return (bit >= 0) & (ndone > 0)

    def body_fn(state):
        bit, p, hitcand, _ = state
        cand = p | (jnp.uint32(1) << jnp.uint32(bit))
        mf = jnp.where(u >= cand, 1.0, 0.0)
        c = jax.lax.dot_general(
            ones_q, mf, (((1,), (0,)), ((), ())),
            preferred_element_type=jnp.float32)
        p = jnp.where(c >= kf, cand, p)
        hit = (c == kf) & (hitcand == 0)
        hitcand = jnp.where(hit, cand, hitcand)
        ndone = jnp.sum(jnp.where(hitcand == 0, 1.0, 0.0))
        return bit - 1, p, hitcand, ndone

    _, p_final, hitcand, ndone = jax.lax.while_loop(
        cond_fn, body_fn, (31, zero_row, zero_row, jnp.float32(1.0)))

    def no_ties(_):
        return jnp.where(u >= hitcand, 1.0, 0.0)

    def with_ties(_):
        thr = jnp.where(hitcand == 0, p_final, hitcand)
        gt = jnp.where(u > thr, 1.0, 0.0)
        need = k - jnp.sum(gt, axis=0, keepdims=True).astype(jnp.int32)
        idx = jax.lax.broadcasted_iota(jnp.int32, u.shape, 0)
        eq = (u == thr)

        def idx_step(i, p):
            cand = p + (1 << (9 - i))
            sel = jnp.where(eq & (idx < cand), 1.0, 0.0)
            c = jnp.sum(sel, axis=0, keepdims=True).astype(jnp.int32)
            return jnp.where(c <= need, cand, p)

        pidx = jax.lax.fori_loop(0, 10, idx_step, jnp.zeros_like(k))
        tie_mask = gt + jnp.where(eq & (idx < pidx), 1.0, 0.0)
        exact = jnp.where(u >= hitcand, 1.0, 0.0)
        return jnp.where(hitcand == 0, tie_mask, exact)

    k_hot = jax.lax.cond(ndone == 0, no_ties, with_ties, operand=None)

    out_t = jax.lax.dot_general(
        cbw_ref[...], k_hot, (((1,), (0,)), ((), ())),
        preferred_element_type=jnp.float32,
    )
    out_ref[...] = out_t.T


@jax.jit
def kernel(x, codebook_weight, k_decider_weight):
    grid = (_B // _BLK,)
    return pl.pallas_call(
        _fused_kernel,
        grid=grid,
        in_specs=[
            pl.BlockSpec((_BLK, _Q), lambda i: (i, 0)),
            pl.BlockSpec((_E, _Q), lambda i: (0, 0)),
            pl.BlockSpec((_E, _Q), lambda i: (0, 0)),
        ],
        out_specs=pl.BlockSpec((_BLK, _E), lambda i: (i, 0)),
        out_shape=jax.ShapeDtypeStruct((_B, _E), jnp.float32),
    )(x, k_decider_weight, codebook_weight)
